# Initial kernel scaffold; baseline (speedup 1.0000x reference)
import jax
import jax.numpy as jnp
from jax.experimental import pallas as pl


def kernel(x, edge_index, W1, b1, W2, b2, W3, b3, Wc, bc):
    raise NotImplementedError("write your pallas kernel here")


# speedup vs baseline: 7.9366x; 7.9366x over previous
import functools

import jax
import jax.numpy as jnp
from jax import lax
from jax.experimental import pallas as pl
from jax.experimental.pallas import tpu as pltpu
from jax.experimental.pallas import tpu_sc as plsc

N = 10000
D = 128
NA = 10112
E = 320000
EP = 323584
RCHUNKS = EP // 128
RPT = RCHUNKS // 32
SLICE = NA // 16
BN = 1000


def _deg_body(dsts_hbm, ones_hbm, zeros_hbm, out_hbm, didx_v, ones_v, acc_sh):
    c = lax.axis_index("c")
    s = lax.axis_index("s")
    t0 = s * SLICE
    pltpu.sync_copy(zeros_hbm, acc_sh.at[pl.ds(t0, SLICE)])
    pltpu.sync_copy(ones_hbm, ones_v)
    pltpu.sync_copy(dsts_hbm.at[c * 16 + s], didx_v)
    plsc.subcore_barrier()

    def body(j, _):
        pltpu.sync_copy(ones_v, acc_sh.at[didx_v.at[j]], add=True)
        return 0

    lax.fori_loop(0, RPT, body, 0)
    plsc.subcore_barrier()
    pltpu.sync_copy(acc_sh.at[pl.ds(t0, SLICE)], out_hbm.at[c, pl.ds(t0, SLICE)])


def _agg_body(table_hbm, srcs_hbm, dsts_hbm, zeros_hbm, out_hbm,
              sidx_v, didx_v, rows_v, acc_sh, sem):
    c = lax.axis_index("c")
    s = lax.axis_index("s")
    t0 = s * SLICE
    pltpu.sync_copy(zeros_hbm, acc_sh.at[pl.ds(t0, SLICE)])
    w = c * 16 + s
    pltpu.sync_copy(srcs_hbm.at[w], sidx_v)
    pltpu.sync_copy(dsts_hbm.at[w], didx_v)
    plsc.subcore_barrier()

    def body(j, _):
        pltpu.async_copy(table_hbm.at[sidx_v.at[j]], rows_v, sem).wait()
        pltpu.sync_copy(rows_v, acc_sh.at[didx_v.at[j]], add=True)
        return 0

    lax.fori_loop(0, RPT, body, 0)
    plsc.subcore_barrier()
    pltpu.sync_copy(acc_sh.at[pl.ds(t0, SLICE)], out_hbm.at[c, pl.ds(t0, SLICE)])


@functools.lru_cache(maxsize=1)
def _sc_calls():
    mesh = plsc.VectorSubcoreMesh(core_axis_name="c", subcore_axis_name="s",
                                  num_cores=2, num_subcores=16)
    deg_call = pl.kernel(
        _deg_body,
        out_type=jax.ShapeDtypeStruct((2, NA, 128), jnp.float32),
        mesh=mesh,
        scratch_types=[
            pltpu.VMEM((RPT, 128), jnp.int32),
            pltpu.VMEM((128, 128), jnp.float32),
            pltpu.VMEM_SHARED((NA, 128), jnp.float32),
        ],
    )
    agg_call = pl.kernel(
        _agg_body,
        out_type=jax.ShapeDtypeStruct((2, NA, 128), jnp.float32),
        mesh=mesh,
        scratch_types=[
            pltpu.VMEM((RPT, 128), jnp.int32),
            pltpu.VMEM((RPT, 128), jnp.int32),
            pltpu.VMEM((128, 128), jnp.float32),
            pltpu.VMEM_SHARED((NA, 128), jnp.float32),
            pltpu.SemaphoreType.DMA,
        ],
    )
    return deg_call, agg_call


def _tc1_body(degp_ref, x_ref, xs_ref, dinv_ref):
    deg = degp_ref[0, :, 0] + degp_ref[1, :, 0] + 1.0
    dinv = lax.rsqrt(deg)[:, None]
    dinv_b = jnp.broadcast_to(dinv, (BN, 128))
    dinv_ref[...] = dinv_b
    xs_ref[...] = x_ref[...] * dinv_b


def _tc2_body(p_ref, xs_ref, dinv_ref, w1_ref, b1_ref, w2_ref, out_ref):
    dinv = dinv_ref[...]
    t = dinv * (p_ref[0] + p_ref[1] + xs_ref[...])
    h1 = jnp.maximum(
        jnp.dot(t, w1_ref[...], preferred_element_type=jnp.float32)
        + b1_ref[...], 0.0)
    m2 = jnp.dot(h1, w2_ref[...], preferred_element_type=jnp.float32)
    for cc in range(4):
        out_ref[cc] = m2[:, cc * 128:(cc + 1) * 128] * dinv


def _tc3_body(p0_ref, p1_ref, p2_ref, p3_ref, m2s_ref, dinv_ref,
              b2_ref, w3_ref, out_ref):
    dinv = dinv_ref[...]
    m3 = jnp.zeros((BN, 128), dtype=jnp.float32)
    for cc, p_ref in enumerate((p0_ref, p1_ref, p2_ref, p3_ref)):
        t = dinv * (p_ref[0] + p_ref[1] + m2s_ref[cc])
        h2 = jnp.maximum(t + b2_ref[:, cc * 128:(cc + 1) * 128], 0.0)
        m3 = m3 + jnp.dot(h2, w3_ref[pl.ds(cc * 128, 128), :],
                          preferred_element_type=jnp.float32)
    out_ref[...] = m3 * dinv


def _tc4_body(p_ref, m3s_ref, dinv_ref, b3_ref, wc_ref, bc_ref, out_ref):
    dinv = dinv_ref[...]
    t = dinv * (p_ref[0] + p_ref[1] + m3s_ref[...])
    h3 = jnp.maximum(t + b3_ref[...], 0.0)
    out_ref[...] = (jnp.dot(h3, wc_ref[...], preferred_element_type=jnp.float32)
                    + bc_ref[...])


def _row_spec(width):
    return pl.BlockSpec((BN, width), lambda i: (i, 0))


def _part_spec(width):
    return pl.BlockSpec((2, BN, width), lambda i: (0, i, 0))


def _full_spec(a, b):
    return pl.BlockSpec((a, b), lambda i: (0, 0))


def kernel(x, edge_index, W1, b1, W2, b2, W3, b3, Wc, bc):
    f32 = jnp.float32
    src = jnp.concatenate(
        [edge_index[0], jnp.zeros((EP - E,), jnp.int32)]).reshape(32, RPT, 128)
    dst = jnp.concatenate(
        [edge_index[1], jnp.full((EP - E,), N, jnp.int32)]).reshape(32, RPT, 128)
    zeros = jnp.zeros((SLICE, 128), f32)
    ones128 = jnp.ones((128, 128), f32)

    _deg_call, _agg_call = _sc_calls()
    degp = _deg_call(dst, ones128, zeros)

    grid = (N // BN,)
    xs, dinv_b = pl.pallas_call(
        _tc1_body,
        grid=grid,
        in_specs=[_part_spec(128), _row_spec(128)],
        out_specs=[_row_spec(128), _row_spec(128)],
        out_shape=[jax.ShapeDtypeStruct((N, 128), f32),
                   jax.ShapeDtypeStruct((N, 128), f32)],
    )(degp, x)

    p1 = _agg_call(xs, src, dst, zeros)

    m2s = pl.pallas_call(
        _tc2_body,
        grid=grid,
        in_specs=[_part_spec(128), _row_spec(128), _row_spec(128),
                  _full_spec(128, 1024), _full_spec(1, 1024),
                  _full_spec(1024, 512)],
        out_specs=pl.BlockSpec((4, BN, 128), lambda i: (0, i, 0)),
        out_shape=jax.ShapeDtypeStruct((4, N, 128), f32),
    )(p1, xs, dinv_b, W1, b1.reshape(1, 1024), W2)

    p2 = [_agg_call(m2s[cc], src, dst, zeros) for cc in range(4)]

    m3s = pl.pallas_call(
        _tc3_body,
        grid=grid,
        in_specs=[_part_spec(128)] * 4
                 + [pl.BlockSpec((4, BN, 128), lambda i: (0, i, 0)),
                    _row_spec(128), _full_spec(1, 512), _full_spec(512, 128)],
        out_specs=_row_spec(128),
        out_shape=jax.ShapeDtypeStruct((N, 128), f32),
    )(*p2, m2s, dinv_b, b2.reshape(1, 512), W3)

    p3 = _agg_call(m3s, src, dst, zeros)

    out = pl.pallas_call(
        _tc4_body,
        grid=grid,
        in_specs=[_part_spec(128), _row_spec(128), _row_spec(128),
                  _full_spec(1, 128), _full_spec(128, 40), _full_spec(1, 40)],
        out_specs=pl.BlockSpec((BN, 40), lambda i: (i, 0)),
        out_shape=jax.ShapeDtypeStruct((N, 40), f32),
    )(p3, m3s, dinv_b, b3.reshape(1, 128), Wc, bc.reshape(1, 40))
    return out
